# baseline (speedup 1.0000x reference)
import jax
import jax.numpy as jnp
from jax import lax
from jax.experimental import pallas as pl
from jax.experimental.pallas import tpu as pltpu
from jax.experimental.pallas import tpu_sc as plsc

N = 8192
K = 8
NC = 2
NS = 16
L = 16
CHUNK = N // NS
NV = CHUNK // L
RW = 2 * L

NEG = float("-inf")
BIG = 0x3FFFFFFF

_DNUMS = lax.GatherDimensionNumbers(
    offset_dims=(), collapsed_slice_dims=(0,), start_index_map=(0,))


def _shuf(v, idx):
    return lax.gather(v, idx[:, None], _DNUMS, (1,),
                      mode=lax.GatherScatterMode.PROMISE_IN_BOUNDS)


def _bfly(v, op, iota):
    for k in (1, 2, 4, 8):
        v = op(v, _shuf(v, iota ^ k))
    return v


def _body_sc(x1_hbm, x2_hbm, t1_hbm, t2_hbm,
             ab_v, comb_sh, comb_loc, row_c, row_t, anchor_v, anchor_f,
             sem0, sem1, sem2, sem3):
    c = lax.axis_index("c")
    s = lax.axis_index("s")
    base = s * CHUNK
    coff = 0
    iota = lax.iota(jnp.int32, L)
    negv = jnp.full((L,), NEG, jnp.float32)
    lane0 = iota == 0

    ca = pltpu.async_copy(t1_hbm, anchor_v.at[pl.ds(0, K)], sem0)
    cb = pltpu.async_copy(t2_hbm, anchor_v.at[pl.ds(K, K)], sem1)
    c1 = pltpu.async_copy(x1_hbm.at[pl.ds(0, K)], anchor_f.at[pl.ds(0, K)],
                          sem2)
    c2 = pltpu.async_copy(x2_hbm.at[pl.ds(0, K)], anchor_f.at[pl.ds(K, K)],
                          sem3)

    @pl.when(c == 0)
    def _in1():
        pltpu.sync_copy(x1_hbm.at[pl.ds(base, CHUNK)],
                        ab_v.at[pl.ds(0, CHUNK)])

    @pl.when(c == 1)
    def _in2():
        pltpu.sync_copy(x2_hbm.at[pl.ds(base, CHUNK)],
                        ab_v.at[pl.ds(0, CHUNK)])

    ca.wait()
    cb.wait()
    c1.wait()
    c2.wait()

    NG = NV // 4
    gm, gx = [], []
    for g in range(NG):
        mg = jnp.full((L,), NEG, jnp.float32)
        xg = jnp.full((L,), BIG, jnp.int32)
        for u in range(4):
            off = 64 * g + L * u
            v = ab_v[pl.ds(coff + off, L)]
            m = v > mg
            mg = jnp.where(m, v, mg)
            xg = jnp.where(m, iota + off, xg)
        gm.append(mg)
        gx.append(xg)

    res_v = jnp.full((L,), NEG, jnp.float32)
    res_i = jnp.full((L,), BIG, jnp.int32)
    for r in range(K):
        bv = jnp.full((L,), NEG, jnp.float32)
        bi = jnp.full((L,), BIG, jnp.int32)
        for g in range(NG):
            m = gm[g] > bv
            bv = jnp.where(m, gm[g], bv)
            bi = jnp.where(m, gx[g], bi)
        gv = _bfly(bv, jnp.maximum, iota)
        gi = _bfly(jnp.where(bv == gv, bi, BIG), jnp.minimum, iota)
        res_v = jnp.where(iota == r, gv, res_v)
        res_i = jnp.where(iota == r, gi, res_i)
        if r == K - 1:
            break
        plsc.store_scatter(ab_v, [gi + coff], negv, mask=lane0)
        gb = (gi // 64) * 64
        mg = jnp.full((L,), NEG, jnp.float32)
        xg = jnp.full((L,), BIG, jnp.int32)
        for u in range(4):
            li = gb + L * u + iota
            v = plsc.load_gather(ab_v, [li + coff])
            m = v > mg
            mg = jnp.where(m, v, mg)
            xg = jnp.where(m, li, xg)
        gm = [jnp.where(gb == 64 * g, mg, gm[g]) for g in range(NG)]
        gx = [jnp.where(gb == 64 * g, xg, gx[g]) for g in range(NG)]

    row_c[pl.ds(0, L)] = res_v
    row_c[pl.ds(L, L)] = plsc.bitcast(res_i + base, jnp.float32)
    pltpu.sync_copy(row_c, comb_sh.at[pl.ds(s * RW, RW)])
    plsc.subcore_barrier()
    pltpu.sync_copy(comb_sh, comb_loc)

    @pl.when(s == 0)
    def _merge():
        def merge_step(r, resm):
            def msweep(j, sc):
                bv, bi, bp = sc
                for u in range(4):
                    jj = 4 * j + u
                    v = comb_loc[pl.ds(RW * jj, L)]
                    gx = plsc.bitcast(comb_loc[pl.ds(RW * jj + L, L)],
                                      jnp.int32)
                    m = v > bv
                    bv = jnp.where(m, v, bv)
                    bi = jnp.where(m, gx, bi)
                    bp = jnp.where(m, iota + RW * jj, bp)
                return bv, bi, bp

            bv, bi, bp = lax.fori_loop(
                0, NS // 4, msweep,
                (jnp.full((L,), NEG, jnp.float32),
                 jnp.full((L,), BIG, jnp.int32),
                 jnp.full((L,), BIG, jnp.int32)))
            gv = _bfly(bv, jnp.maximum, iota)
            vm = bv == gv
            gi = _bfly(jnp.where(vm, bi, BIG), jnp.minimum, iota)
            resm = jnp.where(iota == r, gi, resm)
            gp = _bfly(jnp.where(vm & (bi == gi), bp, BIG), jnp.minimum,
                       iota)
            plsc.store_scatter(comb_loc, [gp], negv, mask=lane0)
            return resm

        resm = lax.fori_loop(0, K, merge_step, jnp.zeros((L,), jnp.int32))
        row_t[...] = resm

        @pl.when(c == 0)
        def _w1():
            pltpu.sync_copy(row_t.at[pl.ds(0, K)], t1_hbm)

        @pl.when(c == 1)
        def _w2():
            pltpu.sync_copy(row_t.at[pl.ds(0, K)], t2_hbm)


def _body_tc(x1_ref, x2_ref, w1_ref, w2_ref):
    for x_ref, w_ref in ((x1_ref, w1_ref), (x2_ref, w2_ref)):
        x = x_ref[...]
        e = jnp.exp(x - jnp.max(x))
        w_ref[...] = e * (1.0 / jnp.sum(e))


@jax.jit
def _run(x1, x2):
    topk = pl.kernel(
        _body_sc,
        out_type=[jax.ShapeDtypeStruct((K,), jnp.int32),
                  jax.ShapeDtypeStruct((K,), jnp.int32)],
        mesh=plsc.VectorSubcoreMesh(
            core_axis_name="c", subcore_axis_name="s",
            num_cores=NC, num_subcores=NS),
        scratch_types=[
            pltpu.VMEM((CHUNK,), jnp.float32),
            pltpu.VMEM_SHARED((NS * RW,), jnp.float32),
            pltpu.VMEM((NS * RW,), jnp.float32),
            pltpu.VMEM((RW,), jnp.float32),
            pltpu.VMEM((L,), jnp.int32),
            pltpu.VMEM((2 * K,), jnp.int32),
            pltpu.VMEM((2 * K,), jnp.float32),
            pltpu.SemaphoreType.DMA,
            pltpu.SemaphoreType.DMA,
            pltpu.SemaphoreType.DMA,
            pltpu.SemaphoreType.DMA,
        ],
        compiler_params=pltpu.CompilerParams(needs_layout_passes=False),
        name="top8_sc",
    )
    w1, w2 = pl.pallas_call(
        _body_tc,
        out_shape=[jax.ShapeDtypeStruct((N,), jnp.float32),
                   jax.ShapeDtypeStruct((N,), jnp.float32)],
        name="softmax_tc",
    )(x1, x2)
    x1b, x2b, _, _ = lax.optimization_barrier((x1, x2, w1, w2))
    t1, t2 = topk(x1b, x2b)
    return w1, w2, t1, t2


def kernel(fc1_logits, fc2_logits):
    return _run(fc1_logits, fc2_logits)


# speedup vs baseline: 1.0592x; 1.0592x over previous
import jax
import jax.numpy as jnp
from jax import lax
from jax.experimental import pallas as pl
from jax.experimental.pallas import tpu as pltpu
from jax.experimental.pallas import tpu_sc as plsc

N = 8192
K = 8
NC = 2
NS = 16
L = 16
CHUNK = N // NS
NV = CHUNK // L
RW = 2 * L

NEG = float("-inf")
BIG = 0x3FFFFFFF

_DNUMS = lax.GatherDimensionNumbers(
    offset_dims=(), collapsed_slice_dims=(0,), start_index_map=(0,))


def _shuf(v, idx):
    return lax.gather(v, idx[:, None], _DNUMS, (1,),
                      mode=lax.GatherScatterMode.PROMISE_IN_BOUNDS)


def _bfly(v, op, iota):
    for k in (1, 2, 4, 8):
        v = op(v, _shuf(v, iota ^ k))
    return v


def _body_sc(x1_hbm, x2_hbm, t1_hbm, t2_hbm,
             ab_v, comb_sh, comb_loc, row_c, row_t, anchor_v, anchor_f,
             sem0, sem1, sem2, sem3):
    c = lax.axis_index("c")
    s = lax.axis_index("s")
    base = s * CHUNK
    coff = 0
    iota = lax.iota(jnp.int32, L)
    negv = jnp.full((L,), NEG, jnp.float32)
    lane0 = iota == 0

    ca = pltpu.async_copy(t1_hbm, anchor_v.at[pl.ds(0, K)], sem0)
    cb = pltpu.async_copy(t2_hbm, anchor_v.at[pl.ds(K, K)], sem1)
    c1 = pltpu.async_copy(x1_hbm.at[pl.ds(0, K)], anchor_f.at[pl.ds(0, K)],
                          sem2)
    c2 = pltpu.async_copy(x2_hbm.at[pl.ds(0, K)], anchor_f.at[pl.ds(K, K)],
                          sem3)

    @pl.when(c == 0)
    def _in1():
        pltpu.sync_copy(x1_hbm.at[pl.ds(base, CHUNK)],
                        ab_v.at[pl.ds(0, CHUNK)])

    @pl.when(c == 1)
    def _in2():
        pltpu.sync_copy(x2_hbm.at[pl.ds(base, CHUNK)],
                        ab_v.at[pl.ds(0, CHUNK)])

    ca.wait()
    cb.wait()
    c1.wait()
    c2.wait()

    def round_step(r, carry):
        res_v, res_i = carry

        def sweep(j, sc):
            bv, bi = sc
            for u in range(4):
                off = L * (4 * j + u)
                v = ab_v[pl.ds(coff + off, L)]
                m = v > bv
                bv = jnp.where(m, v, bv)
                bi = jnp.where(m, iota + off, bi)
            return bv, bi

        bv, bi = lax.fori_loop(
            0, NV // 4, sweep,
            (jnp.full((L,), NEG, jnp.float32),
             jnp.full((L,), BIG, jnp.int32)))
        gv = _bfly(bv, jnp.maximum, iota)
        gi = _bfly(jnp.where(bv == gv, bi, BIG), jnp.minimum, iota)
        res_v = jnp.where(iota == r, gv, res_v)
        res_i = jnp.where(iota == r, gi, res_i)
        plsc.store_scatter(ab_v, [gi + coff], negv, mask=lane0)
        return res_v, res_i

    res_v, res_i = lax.fori_loop(
        0, K, round_step,
        (jnp.full((L,), NEG, jnp.float32), jnp.full((L,), BIG, jnp.int32)))

    row_c[pl.ds(0, L)] = res_v
    row_c[pl.ds(L, L)] = plsc.bitcast(res_i + base, jnp.float32)
    pltpu.sync_copy(row_c, comb_sh.at[pl.ds(s * RW, RW)])
    plsc.subcore_barrier()
    pltpu.sync_copy(comb_sh, comb_loc)

    @pl.when(s == 0)
    def _merge():
        def merge_step(r, resm):
            def msweep(j, sc):
                bv, bi, bp = sc
                for u in range(4):
                    jj = 4 * j + u
                    v = comb_loc[pl.ds(RW * jj, L)]
                    gx = plsc.bitcast(comb_loc[pl.ds(RW * jj + L, L)],
                                      jnp.int32)
                    m = v > bv
                    bv = jnp.where(m, v, bv)
                    bi = jnp.where(m, gx, bi)
                    bp = jnp.where(m, iota + RW * jj, bp)
                return bv, bi, bp

            bv, bi, bp = lax.fori_loop(
                0, NS // 4, msweep,
                (jnp.full((L,), NEG, jnp.float32),
                 jnp.full((L,), BIG, jnp.int32),
                 jnp.full((L,), BIG, jnp.int32)))
            gv = _bfly(bv, jnp.maximum, iota)
            vm = bv == gv
            gi = _bfly(jnp.where(vm, bi, BIG), jnp.minimum, iota)
            resm = jnp.where(iota == r, gi, resm)
            gp = _bfly(jnp.where(vm & (bi == gi), bp, BIG), jnp.minimum,
                       iota)
            plsc.store_scatter(comb_loc, [gp], negv, mask=lane0)
            return resm

        resm = lax.fori_loop(0, K, merge_step, jnp.zeros((L,), jnp.int32))
        row_t[...] = resm

        @pl.when(c == 0)
        def _w1():
            pltpu.sync_copy(row_t.at[pl.ds(0, K)], t1_hbm)

        @pl.when(c == 1)
        def _w2():
            pltpu.sync_copy(row_t.at[pl.ds(0, K)], t2_hbm)


def _body_tc(x1_ref, x2_ref, w1_ref, w2_ref):
    for x_ref, w_ref in ((x1_ref, w1_ref), (x2_ref, w2_ref)):
        x = x_ref[...]
        e = jnp.exp(x - jnp.max(x))
        w_ref[...] = e * (1.0 / jnp.sum(e))


@jax.jit
def _run(x1, x2):
    topk = pl.kernel(
        _body_sc,
        out_type=[jax.ShapeDtypeStruct((K,), jnp.int32),
                  jax.ShapeDtypeStruct((K,), jnp.int32)],
        mesh=plsc.VectorSubcoreMesh(
            core_axis_name="c", subcore_axis_name="s",
            num_cores=NC, num_subcores=NS),
        scratch_types=[
            pltpu.VMEM((CHUNK,), jnp.float32),
            pltpu.VMEM_SHARED((NS * RW,), jnp.float32),
            pltpu.VMEM((NS * RW,), jnp.float32),
            pltpu.VMEM((RW,), jnp.float32),
            pltpu.VMEM((L,), jnp.int32),
            pltpu.VMEM((2 * K,), jnp.int32),
            pltpu.VMEM((2 * K,), jnp.float32),
            pltpu.SemaphoreType.DMA,
            pltpu.SemaphoreType.DMA,
            pltpu.SemaphoreType.DMA,
            pltpu.SemaphoreType.DMA,
        ],
        compiler_params=pltpu.CompilerParams(needs_layout_passes=False),
        name="top8_sc",
    )
    w1, w2 = pl.pallas_call(
        _body_tc,
        out_shape=[jax.ShapeDtypeStruct((N,), jnp.float32),
                   jax.ShapeDtypeStruct((N,), jnp.float32)],
        name="softmax_tc",
    )(x1, x2)
    t1, t2 = topk(x1, x2)
    return w1, w2, t1, t2


def kernel(fc1_logits, fc2_logits):
    return _run(fc1_logits, fc2_logits)


# speedup vs baseline: 1.0607x; 1.0014x over previous
import jax
import jax.numpy as jnp
from jax import lax
from jax.experimental import pallas as pl
from jax.experimental.pallas import tpu as pltpu
from jax.experimental.pallas import tpu_sc as plsc

N = 8192
K = 8
NC = 2
NS = 16
L = 16
CHUNK = N // NS
NV = CHUNK // L
RW = 2 * L

NEG = float("-inf")
BIG = 0x3FFFFFFF

_DNUMS = lax.GatherDimensionNumbers(
    offset_dims=(), collapsed_slice_dims=(0,), start_index_map=(0,))


def _shuf(v, idx):
    return lax.gather(v, idx[:, None], _DNUMS, (1,),
                      mode=lax.GatherScatterMode.PROMISE_IN_BOUNDS)


def _bfly(v, op, iota):
    for k in (1, 2, 4, 8):
        v = op(v, _shuf(v, iota ^ k))
    return v


def _body_sc(x1_hbm, x2_hbm, t1_hbm, t2_hbm,
             ab_v, comb_sh, comb_loc, row_c, row_t, anchor_v, anchor_f,
             sem0, sem1, sem2, sem3):
    c = lax.axis_index("c")
    s = lax.axis_index("s")
    base = s * CHUNK
    coff = 0
    iota = lax.iota(jnp.int32, L)
    negv = jnp.full((L,), NEG, jnp.float32)
    lane0 = iota == 0

    ca = pltpu.async_copy(t1_hbm, anchor_v.at[pl.ds(0, K)], sem0)
    cb = pltpu.async_copy(t2_hbm, anchor_v.at[pl.ds(K, K)], sem1)
    c1 = pltpu.async_copy(x1_hbm.at[pl.ds(0, K)], anchor_f.at[pl.ds(0, K)],
                          sem2)
    c2 = pltpu.async_copy(x2_hbm.at[pl.ds(0, K)], anchor_f.at[pl.ds(K, K)],
                          sem3)

    @pl.when(c == 0)
    def _in1():
        pltpu.sync_copy(x1_hbm.at[pl.ds(base, CHUNK)],
                        ab_v.at[pl.ds(0, CHUNK)])

    @pl.when(c == 1)
    def _in2():
        pltpu.sync_copy(x2_hbm.at[pl.ds(base, CHUNK)],
                        ab_v.at[pl.ds(0, CHUNK)])

    ca.wait()
    cb.wait()
    c1.wait()
    c2.wait()

    def round_step(r, carry):
        res_v, res_i = carry

        def sweep(j, sc):
            bv, bi = sc
            for u in range(4):
                off = L * (4 * j + u)
                v = ab_v[pl.ds(coff + off, L)]
                m = v > bv
                bv = jnp.where(m, v, bv)
                bi = jnp.where(m, iota + off, bi)
            return bv, bi

        bv, bi = lax.fori_loop(
            0, NV // 4, sweep,
            (jnp.full((L,), NEG, jnp.float32),
             jnp.full((L,), BIG, jnp.int32)))
        gv = _bfly(bv, jnp.maximum, iota)
        gi = _bfly(jnp.where(bv == gv, bi, BIG), jnp.minimum, iota)
        res_v = jnp.where(iota == r, gv, res_v)
        res_i = jnp.where(iota == r, gi, res_i)
        plsc.store_scatter(ab_v, [gi + coff], negv, mask=lane0)
        return res_v, res_i

    res_v, res_i = lax.fori_loop(
        0, K, round_step,
        (jnp.full((L,), NEG, jnp.float32), jnp.full((L,), BIG, jnp.int32)))

    row_c[pl.ds(0, L)] = res_v
    row_c[pl.ds(L, L)] = plsc.bitcast(res_i + base, jnp.float32)
    pltpu.sync_copy(row_c, comb_sh.at[pl.ds(s * RW, RW)])
    plsc.subcore_barrier()
    pltpu.sync_copy(comb_sh, comb_loc)

    @pl.when(s == 0)
    def _merge():
        def merge_step(r, resm):
            def msweep(j, sc):
                bv, bi, bp = sc
                for u in range(4):
                    jj = 4 * j + u
                    v = comb_loc[pl.ds(RW * jj, L)]
                    gx = plsc.bitcast(comb_loc[pl.ds(RW * jj + L, L)],
                                      jnp.int32)
                    m = v > bv
                    bv = jnp.where(m, v, bv)
                    bi = jnp.where(m, gx, bi)
                    bp = jnp.where(m, iota + RW * jj, bp)
                return bv, bi, bp

            bv, bi, bp = lax.fori_loop(
                0, NS // 4, msweep,
                (jnp.full((L,), NEG, jnp.float32),
                 jnp.full((L,), BIG, jnp.int32),
                 jnp.full((L,), BIG, jnp.int32)))
            gv = _bfly(bv, jnp.maximum, iota)
            vm = bv == gv
            gi = _bfly(jnp.where(vm, bi, BIG), jnp.minimum, iota)
            resm = jnp.where(iota == r, gi, resm)
            gp = _bfly(jnp.where(vm & (bi == gi), bp, BIG), jnp.minimum,
                       iota)
            plsc.store_scatter(comb_loc, [gp], negv, mask=lane0)
            return resm

        resm = lax.fori_loop(0, K, merge_step, jnp.zeros((L,), jnp.int32))
        row_t[...] = resm

        @pl.when(c == 0)
        def _w1():
            pltpu.sync_copy(row_t.at[pl.ds(0, K)], t1_hbm)

        @pl.when(c == 1)
        def _w2():
            pltpu.sync_copy(row_t.at[pl.ds(0, K)], t2_hbm)


def _body_tc(x1_ref, x2_ref, w1_ref, w2_ref):
    for x_ref, w_ref in ((x1_ref, w1_ref), (x2_ref, w2_ref)):
        x = x_ref[...]
        e = jnp.exp(x - jnp.max(x))
        w_ref[...] = e * (1.0 / jnp.sum(e))


@jax.jit
def _run(x1, x2):
    topk = pl.kernel(
        _body_sc,
        out_type=[jax.ShapeDtypeStruct((K,), jnp.int32),
                  jax.ShapeDtypeStruct((K,), jnp.int32)],
        mesh=plsc.VectorSubcoreMesh(
            core_axis_name="c", subcore_axis_name="s",
            num_cores=NC, num_subcores=NS),
        scratch_types=[
            pltpu.VMEM((CHUNK,), jnp.float32),
            pltpu.VMEM_SHARED((NS * RW,), jnp.float32),
            pltpu.VMEM((NS * RW,), jnp.float32),
            pltpu.VMEM((RW,), jnp.float32),
            pltpu.VMEM((L,), jnp.int32),
            pltpu.VMEM((2 * K,), jnp.int32),
            pltpu.VMEM((2 * K,), jnp.float32),
            pltpu.SemaphoreType.DMA,
            pltpu.SemaphoreType.DMA,
            pltpu.SemaphoreType.DMA,
            pltpu.SemaphoreType.DMA,
        ],
        compiler_params=pltpu.CompilerParams(needs_layout_passes=False),
        name="top8_sc",
    )
    w1, w2 = pl.pallas_call(
        _body_tc,
        out_shape=[jax.ShapeDtypeStruct((N,), jnp.float32),
                   jax.ShapeDtypeStruct((N,), jnp.float32)],
        name="softmax_tc",
    )(x1, x2)
    x1b, x2b, _, _ = lax.optimization_barrier((x1, x2, w1, w2))
    t1, t2 = topk(x1b, x2b)
    return w1, w2, t1, t2


def kernel(fc1_logits, fc2_logits):
    return _run(fc1_logits, fc2_logits)
